# baseline (speedup 1.0000x reference)
import functools

import jax
import jax.numpy as jnp
from jax import lax
from jax.experimental import pallas as pl
from jax.experimental.pallas import tpu as pltpu
from jax.experimental.pallas import tpu_sc as plsc

D = 64
NW = 32
CHUNK = 128


def _build_gather(ntok: int):
    per_w = ntok // NW
    nchunk = per_w // CHUNK
    npair = nchunk // 2
    mesh = plsc.VectorSubcoreMesh(core_axis_name="c", subcore_axis_name="s")

    @functools.partial(
        pl.kernel,
        mesh=mesh,
        compiler_params=pltpu.CompilerParams(use_tc_tiling_on_sc=False),
        out_type=jax.ShapeDtypeStruct((ntok, D), jnp.float32),
        scratch_types=[
            pltpu.VMEM((nchunk, CHUNK), jnp.int32),
            pltpu.VMEM((CHUNK, D), jnp.float32),
            pltpu.VMEM((CHUNK, D), jnp.float32),
            pltpu.SemaphoreType.DMA,
            pltpu.SemaphoreType.DMA,
        ],
    )
    def emb(table_hbm, idx_hbm, out_hbm, idx_v, buf0, buf1, g0, g1):
        wid = lax.axis_index("s") * 2 + lax.axis_index("c")
        base = wid * per_w
        pltpu.sync_copy(idx_hbm.at[wid], idx_v)
        pltpu.async_copy(table_hbm.at[idx_v.at[0]], buf0, g0)

        def body(p, carry):
            ja = 2 * p
            jb = ja + 1
            pltpu.make_async_copy(table_hbm.at[idx_v.at[ja]], buf0, g0).wait()
            pltpu.async_copy(table_hbm.at[idx_v.at[jb]], buf1, g1)
            pltpu.sync_copy(buf0, out_hbm.at[pl.ds(base + ja * CHUNK, CHUNK)])
            pltpu.make_async_copy(table_hbm.at[idx_v.at[jb]], buf1, g1).wait()

            @pl.when(p + 1 < npair)
            def _():
                pltpu.async_copy(table_hbm.at[idx_v.at[jb + 1]], buf0, g0)

            pltpu.sync_copy(buf1, out_hbm.at[pl.ds(base + jb * CHUNK, CHUNK)])
            return carry

        lax.fori_loop(0, npair, body, 0)

    return emb


def kernel(input_ids, attention_mask, table):
    b, s = input_ids.shape
    ntok = b * s
    idx3 = input_ids.reshape(NW, ntok // (NW * CHUNK), CHUNK).astype(jnp.int32)
    out = _build_gather(ntok)(table, idx3)
    return out.reshape(b, s, D), attention_mask
